# Initial kernel scaffold; baseline (speedup 1.0000x reference)
import jax
import jax.numpy as jnp
from jax.experimental import pallas as pl


def kernel(table, input_ids):
    raise NotImplementedError("write your pallas kernel here")


# speedup vs baseline: 9.3867x; 9.3867x over previous
import functools

import jax
import jax.numpy as jnp
from jax import lax
from jax.experimental import pallas as pl
from jax.experimental.pallas import tpu as pltpu
from jax.experimental.pallas import tpu_sc as plsc

VOCAB = 1000000
D = 16
B = 16384
L = 200

_INFO = plsc.get_sparse_core_info()
NC = _INFO.num_cores
NS = _INFO.num_subcores
NW = NC * NS
BPW = B // NW

C = 8
NCHUNK = BPW // C
IW = 100
G = C * L // IW
IROWS = C * L // IW
NBUF = 4

_mesh = plsc.VectorSubcoreMesh(core_axis_name="c", subcore_axis_name="s")


@functools.partial(
    pl.kernel,
    mesh=_mesh,
    out_type=jax.ShapeDtypeStruct((B, D), jnp.float32),
    scratch_types=(
        [pltpu.VMEM((IROWS, IW), jnp.int32) for _ in range(NBUF)]
        + [pltpu.VMEM((C * L, D), jnp.float32) for _ in range(NBUF)]
        + [pltpu.VMEM((BPW, D), jnp.float32)]
        + [pltpu.SemaphoreType.DMA for _ in range(2 * NBUF)]
    ),
    compiler_params=pltpu.CompilerParams(use_tc_tiling_on_sc=False),
)
def _emb_mean(ids_hbm, table_hbm, out_hbm, *refs):
    idx = refs[0:NBUF]
    rows = refs[NBUF:2 * NBUF]
    outf = refs[2 * NBUF]
    isem = refs[2 * NBUF + 1:2 * NBUF + 1 + NBUF]
    gsem = refs[2 * NBUF + 1 + NBUF:]

    cid = lax.axis_index("c")
    sid = lax.axis_index("s")
    wid = sid * NC + cid
    base = wid * BPW

    def fire_idx(ci, par):
        irow = (base + ci * C) * (L // IW)
        pltpu.async_copy(ids_hbm.at[pl.ds(irow, IROWS)], idx[par], isem[par])

    def fire_gather(par):
        pltpu.make_async_copy(
            ids_hbm.at[pl.ds(0, IROWS)], idx[par], isem[par]).wait()
        for g in range(G):
            pltpu.async_copy(
                table_hbm.at[idx[par].at[g]],
                rows[par].at[pl.ds(g * IW, IW)],
                gsem[par])

    def drain_rows(par):
        pltpu.make_async_copy(
            table_hbm.at[pl.ds(0, C * L)], rows[par], gsem[par]).wait()

    scale = jnp.float32(1.0 / L)

    def reduce_store(ci, par):
        rb = rows[par]

        def row_body(r, carry):
            p0 = r * L
            a0 = rb[p0]
            a1 = rb[p0 + 1]
            a2 = rb[p0 + 2]
            a3 = rb[p0 + 3]
            for j in range(4, L, 4):
                a0 = a0 + rb[p0 + j]
                a1 = a1 + rb[p0 + j + 1]
                a2 = a2 + rb[p0 + j + 2]
                a3 = a3 + rb[p0 + j + 3]
            outf[ci * C + r] = ((a0 + a1) + (a2 + a3)) * scale
            return carry

        lax.fori_loop(0, C, row_body, 0)

    for par in range(NBUF):
        fire_idx(par, par)
    for par in range(NBUF):
        fire_gather(par)

    def group_body(cg, carry):
        for par in range(NBUF):
            ci = cg * NBUF + par
            drain_rows(par)

            @pl.when(ci + NBUF < NCHUNK)
            def _():
                fire_idx(ci + NBUF, par)

            reduce_store(ci, par)

            @pl.when(ci + NBUF < NCHUNK)
            def _():
                fire_gather(par)
        return carry

    lax.fori_loop(0, NCHUNK // NBUF, group_body, 0)

    pltpu.sync_copy(outf, out_hbm.at[pl.ds(base, BPW)])


def kernel(table, input_ids):
    ids = input_ids.astype(jnp.int32).reshape(B * L // IW, IW)
    return _emb_mean(ids, table)


# speedup vs baseline: 9.5399x; 1.0163x over previous
import functools

import jax
import jax.numpy as jnp
from jax import lax
from jax.experimental import pallas as pl
from jax.experimental.pallas import tpu as pltpu
from jax.experimental.pallas import tpu_sc as plsc

VOCAB = 1000000
D = 16
B = 16384
L = 200

_INFO = plsc.get_sparse_core_info()
NC = _INFO.num_cores
NS = _INFO.num_subcores
NW = NC * NS
BPW = B // NW

C = 8
NCHUNK = BPW // C
SPLITS = ((0, 104), (104, 96))
NBUF = 4

_mesh = plsc.VectorSubcoreMesh(core_axis_name="c", subcore_axis_name="s")


@functools.partial(
    pl.kernel,
    mesh=_mesh,
    out_type=jax.ShapeDtypeStruct((B, D), jnp.float32),
    scratch_types=(
        [pltpu.VMEM((C, L), jnp.int32) for _ in range(NBUF)]
        + [pltpu.VMEM((C * L, D), jnp.float32) for _ in range(NBUF)]
        + [pltpu.VMEM((BPW, D), jnp.float32)]
        + [pltpu.SemaphoreType.DMA for _ in range(2 * NBUF)]
    ),
    compiler_params=pltpu.CompilerParams(use_tc_tiling_on_sc=False),
)
def _emb_mean(ids_hbm, table_hbm, out_hbm, *refs):
    idx = refs[0:NBUF]
    rows = refs[NBUF:2 * NBUF]
    outf = refs[2 * NBUF]
    isem = refs[2 * NBUF + 1:2 * NBUF + 1 + NBUF]
    gsem = refs[2 * NBUF + 1 + NBUF:]

    cid = lax.axis_index("c")
    sid = lax.axis_index("s")
    wid = sid * NC + cid
    base = wid * BPW

    def fire_idx(ci, par):
        pltpu.async_copy(
            ids_hbm.at[pl.ds(base + ci * C, C)], idx[par], isem[par])

    def fire_gather(par):
        pltpu.make_async_copy(
            ids_hbm.at[pl.ds(0, C)], idx[par], isem[par]).wait()
        for r in range(C):
            for off, width in SPLITS:
                pltpu.async_copy(
                    table_hbm.at[idx[par].at[r].at[pl.ds(off, width)]],
                    rows[par].at[pl.ds(r * L + off, width)],
                    gsem[par])

    def drain_rows(par):
        pltpu.make_async_copy(
            table_hbm.at[pl.ds(0, C * L)], rows[par], gsem[par]).wait()

    scale = jnp.float32(1.0 / L)

    def reduce_store(ci, par):
        rb = rows[par]

        def row_body(r, carry):
            p0 = r * L
            a0 = rb[p0]
            a1 = rb[p0 + 1]
            a2 = rb[p0 + 2]
            a3 = rb[p0 + 3]
            for j in range(4, L, 4):
                a0 = a0 + rb[p0 + j]
                a1 = a1 + rb[p0 + j + 1]
                a2 = a2 + rb[p0 + j + 2]
                a3 = a3 + rb[p0 + j + 3]
            outf[ci * C + r] = ((a0 + a1) + (a2 + a3)) * scale
            return carry

        lax.fori_loop(0, C, row_body, 0)

    for par in range(NBUF):
        fire_idx(par, par)
    for par in range(NBUF):
        fire_gather(par)

    def group_body(cg, carry):
        for par in range(NBUF):
            ci = cg * NBUF + par
            drain_rows(par)

            @pl.when(ci + NBUF < NCHUNK)
            def _():
                fire_idx(ci + NBUF, par)

            reduce_store(ci, par)

            @pl.when(ci + NBUF < NCHUNK)
            def _():
                fire_gather(par)
        return carry

    lax.fori_loop(0, NCHUNK // NBUF, group_body, 0)

    pltpu.sync_copy(outf, out_hbm.at[pl.ds(base, BPW)])


def kernel(table, input_ids):
    return _emb_mean(input_ids.astype(jnp.int32), table)
